# initial kernel scaffold (unmeasured)
import jax
import jax.numpy as jnp
from jax import lax
from jax.experimental import pallas as pl
from jax.experimental.pallas import tpu as pltpu


def kernel(
    x,
):
    def body(*refs):
        pass

    out_shape = jax.ShapeDtypeStruct(..., jnp.float32)
    return pl.pallas_call(body, out_shape=out_shape)(...)


# baseline (device time: 47463 ns/iter reference)
import jax
import jax.numpy as jnp
from jax import lax
from jax.experimental import pallas as pl
from jax.experimental.pallas import tpu as pltpu

N_DEV = 4


def kernel(x):
    _, m, n_full = x.shape
    n_out = n_full // N_DEV

    def body(x_ref, out_ref, comm_ref, send_sems, recv_sems):
        my_x = lax.axis_index("x")
        my_y = lax.axis_index("y")
        my_z = lax.axis_index("z")
        left = (my_z + N_DEV - 1) % N_DEV
        right = (my_z + 1) % N_DEV

        barrier_sem = pltpu.get_barrier_semaphore()
        for nbr in (left, right):
            pl.semaphore_signal(
                barrier_sem,
                inc=1,
                device_id=(my_x, my_y, nbr),
                device_id_type=pl.DeviceIdType.MESH,
            )
        pl.semaphore_wait(barrier_sem, 2)

        c0 = (my_z + N_DEV - 1) % N_DEV
        comm_ref[N_DEV - 1, :, :] = x_ref[
            0, :, pl.ds(c0 * n_out, n_out)
        ].astype(jnp.bfloat16)

        for h in range(N_DEV - 1):
            src_slot = (N_DEV - 1) if h == 0 else h - 1
            rdma = pltpu.make_async_remote_copy(
                src_ref=comm_ref.at[src_slot],
                dst_ref=comm_ref.at[h],
                send_sem=send_sems.at[h],
                recv_sem=recv_sems.at[h],
                device_id=(my_x, my_y, right),
                device_id_type=pl.DeviceIdType.MESH,
            )
            rdma.start()
            rdma.wait()
            c = (my_z + 2 * N_DEV - 2 - h) % N_DEV
            comm_ref[h, :, :] += x_ref[0, :, pl.ds(c * n_out, n_out)].astype(
                jnp.bfloat16
            )

        out_ref[:, :] = comm_ref[N_DEV - 2, :, :]

    return pl.pallas_call(
        body,
        out_shape=jax.ShapeDtypeStruct((m, n_out), jnp.bfloat16),
        in_specs=[pl.BlockSpec(memory_space=pltpu.VMEM)],
        out_specs=pl.BlockSpec(memory_space=pltpu.VMEM),
        scratch_shapes=[
            pltpu.VMEM((N_DEV, m, n_out), jnp.bfloat16),
            pltpu.SemaphoreType.DMA((N_DEV - 1,)),
            pltpu.SemaphoreType.DMA((N_DEV - 1,)),
        ],
        compiler_params=pltpu.CompilerParams(collective_id=0),
    )(x)


# device time: 46444 ns/iter; 1.0219x vs baseline; 1.0219x over previous
import jax
import jax.numpy as jnp
from jax import lax
from jax.experimental import pallas as pl
from jax.experimental.pallas import tpu as pltpu

N_DEV = 4


def kernel(x):
    _, m, n_full = x.shape
    n_out = n_full // N_DEV
    m_half = m // 2

    def body(x_ref, out_ref, cw_ref, ccw_ref, cw_send, cw_recv, ccw_send, ccw_recv):
        my_x = lax.axis_index("x")
        my_y = lax.axis_index("y")
        my_z = lax.axis_index("z")
        left = (my_z + N_DEV - 1) % N_DEV
        right = (my_z + 1) % N_DEV

        barrier_sem = pltpu.get_barrier_semaphore()
        for nbr in (left, right):
            pl.semaphore_signal(
                barrier_sem,
                inc=1,
                device_id=(my_x, my_y, nbr),
                device_id_type=pl.DeviceIdType.MESH,
            )
        pl.semaphore_wait(barrier_sem, 2)

        c0_cw = (my_z + N_DEV - 1) % N_DEV
        c0_ccw = (my_z + 1) % N_DEV
        cw_ref[N_DEV - 1, :, :] = x_ref[
            0, :m_half, pl.ds(c0_cw * n_out, n_out)
        ].astype(jnp.bfloat16)
        ccw_ref[N_DEV - 1, :, :] = x_ref[
            0, m_half:, pl.ds(c0_ccw * n_out, n_out)
        ].astype(jnp.bfloat16)

        rdmas = []
        for h in range(N_DEV - 1):
            src_slot = (N_DEV - 1) if h == 0 else h - 1
            cw = pltpu.make_async_remote_copy(
                src_ref=cw_ref.at[src_slot],
                dst_ref=cw_ref.at[h],
                send_sem=cw_send.at[h],
                recv_sem=cw_recv.at[h],
                device_id=(my_x, my_y, right),
                device_id_type=pl.DeviceIdType.MESH,
            )
            ccw = pltpu.make_async_remote_copy(
                src_ref=ccw_ref.at[src_slot],
                dst_ref=ccw_ref.at[h],
                send_sem=ccw_send.at[h],
                recv_sem=ccw_recv.at[h],
                device_id=(my_x, my_y, left),
                device_id_type=pl.DeviceIdType.MESH,
            )
            cw.start()
            ccw.start()
            rdmas.append((cw, ccw))

            c_cw = (my_z + 2 * N_DEV - 2 - h) % N_DEV
            c_ccw = (my_z + 2 + h) % N_DEV
            cw.wait_recv()
            cw_ref[h, :, :] += x_ref[
                0, :m_half, pl.ds(c_cw * n_out, n_out)
            ].astype(jnp.bfloat16)
            ccw.wait_recv()
            ccw_ref[h, :, :] += x_ref[
                0, m_half:, pl.ds(c_ccw * n_out, n_out)
            ].astype(jnp.bfloat16)

        out_ref[:m_half, :] = cw_ref[N_DEV - 2, :, :]
        out_ref[m_half:, :] = ccw_ref[N_DEV - 2, :, :]

        for cw, ccw in rdmas:
            cw.wait_send()
            ccw.wait_send()

    return pl.pallas_call(
        body,
        out_shape=jax.ShapeDtypeStruct((m, n_out), jnp.bfloat16),
        in_specs=[pl.BlockSpec(memory_space=pltpu.VMEM)],
        out_specs=pl.BlockSpec(memory_space=pltpu.VMEM),
        scratch_shapes=[
            pltpu.VMEM((N_DEV, m_half, n_out), jnp.bfloat16),
            pltpu.VMEM((N_DEV, m_half, n_out), jnp.bfloat16),
            pltpu.SemaphoreType.DMA((N_DEV - 1,)),
            pltpu.SemaphoreType.DMA((N_DEV - 1,)),
            pltpu.SemaphoreType.DMA((N_DEV - 1,)),
            pltpu.SemaphoreType.DMA((N_DEV - 1,)),
        ],
        compiler_params=pltpu.CompilerParams(collective_id=0),
    )(x)


# device time: 30637 ns/iter; 1.5492x vs baseline; 1.5159x over previous
import jax
import jax.numpy as jnp
from jax import lax
from jax.experimental import pallas as pl
from jax.experimental.pallas import tpu as pltpu

NZ = 4
NY = 4


def kernel(x):
    _, m, n_full = x.shape
    n_out = n_full // NZ
    m_slice = m // 8
    m_block = m // NY

    def body(x_ref, out_ref, zsend_ref, zrecv_ref, zsend_sems, zrecv_sems,
             xsend_sem, xrecv_sem, ysend_sems, yrecv_sems):
        my_x = lax.axis_index("x")
        my_y = lax.axis_index("y")
        my_z = lax.axis_index("z")
        s = my_y * 2 + my_x
        row0 = s * m_slice

        barrier_sem = pltpu.get_barrier_semaphore()
        for j in range(1, NZ):
            pl.semaphore_signal(
                barrier_sem, inc=1,
                device_id=(my_x, my_y, (my_z + j) % NZ),
                device_id_type=pl.DeviceIdType.MESH,
            )
        for j in range(1, NY):
            pl.semaphore_signal(
                barrier_sem, inc=1,
                device_id=(my_x, (my_y + j) % NY, my_z),
                device_id_type=pl.DeviceIdType.MESH,
            )
        pl.semaphore_signal(
            barrier_sem, inc=1,
            device_id=(1 - my_x, my_y, my_z),
            device_id_type=pl.DeviceIdType.MESH,
        )
        pl.semaphore_wait(barrier_sem, NZ - 1 + NY - 1 + 1)

        z_rdmas = []
        for j in range(1, NZ):
            c = (my_z + j) % NZ
            zsend_ref[j - 1, :, :] = x_ref[
                0, pl.ds(row0, m_slice), pl.ds(c * n_out, n_out)
            ].astype(jnp.bfloat16)
            rdma = pltpu.make_async_remote_copy(
                src_ref=zsend_ref.at[j - 1],
                dst_ref=zrecv_ref.at[my_z],
                send_sem=zsend_sems.at[j - 1],
                recv_sem=zrecv_sems.at[j - 1],
                device_id=(my_x, my_y, c),
                device_id_type=pl.DeviceIdType.MESH,
            )
            rdma.start()
            z_rdmas.append(rdma)

        zrecv_ref[my_z, :, :] = x_ref[
            0, pl.ds(row0, m_slice), pl.ds(my_z * n_out, n_out)
        ].astype(jnp.bfloat16)

        for j in range(1, NZ):
            recv = pltpu.make_async_remote_copy(
                src_ref=zsend_ref.at[j - 1],
                dst_ref=zrecv_ref.at[(my_z + NZ - j) % NZ],
                send_sem=zsend_sems.at[j - 1],
                recv_sem=zrecv_sems.at[j - 1],
                device_id=(my_x, my_y, my_z),
                device_id_type=pl.DeviceIdType.MESH,
            )
            recv.wait_recv()

        out_ref[pl.ds(row0, m_slice), :] = (
            zrecv_ref[0] + zrecv_ref[1] + zrecv_ref[2] + zrecv_ref[3]
        )

        x_rdma = pltpu.make_async_remote_copy(
            src_ref=out_ref.at[pl.ds(row0, m_slice), :],
            dst_ref=out_ref.at[pl.ds(row0, m_slice), :],
            send_sem=xsend_sem,
            recv_sem=xrecv_sem,
            device_id=(1 - my_x, my_y, my_z),
            device_id_type=pl.DeviceIdType.MESH,
        )
        x_rdma.start()
        partner_row0 = (my_y * 2 + (1 - my_x)) * m_slice
        x_recv = pltpu.make_async_remote_copy(
            src_ref=out_ref.at[pl.ds(row0, m_slice), :],
            dst_ref=out_ref.at[pl.ds(partner_row0, m_slice), :],
            send_sem=xsend_sem,
            recv_sem=xrecv_sem,
            device_id=(1 - my_x, my_y, my_z),
            device_id_type=pl.DeviceIdType.MESH,
        )
        x_recv.wait_recv()

        block_row0 = my_y * m_block
        y_rdmas = []
        for j in range(1, NY):
            ty = (my_y + j) % NY
            rdma = pltpu.make_async_remote_copy(
                src_ref=out_ref.at[pl.ds(block_row0, m_block), :],
                dst_ref=out_ref.at[pl.ds(block_row0, m_block), :],
                send_sem=ysend_sems.at[j - 1],
                recv_sem=yrecv_sems.at[j - 1],
                device_id=(my_x, ty, my_z),
                device_id_type=pl.DeviceIdType.MESH,
            )
            rdma.start()
            y_rdmas.append(rdma)

        for j in range(1, NY):
            src_y = (my_y + NY - j) % NY
            recv = pltpu.make_async_remote_copy(
                src_ref=out_ref.at[pl.ds(block_row0, m_block), :],
                dst_ref=out_ref.at[pl.ds(src_y * m_block, m_block), :],
                send_sem=ysend_sems.at[j - 1],
                recv_sem=yrecv_sems.at[j - 1],
                device_id=(my_x, my_y, my_z),
                device_id_type=pl.DeviceIdType.MESH,
            )
            recv.wait_recv()

        for rdma in z_rdmas:
            rdma.wait_send()
        x_rdma.wait_send()
        for rdma in y_rdmas:
            rdma.wait_send()

    return pl.pallas_call(
        body,
        out_shape=jax.ShapeDtypeStruct((m, n_out), jnp.bfloat16),
        in_specs=[pl.BlockSpec(memory_space=pltpu.VMEM)],
        out_specs=pl.BlockSpec(memory_space=pltpu.VMEM),
        scratch_shapes=[
            pltpu.VMEM((NZ - 1, m_slice, n_out), jnp.bfloat16),
            pltpu.VMEM((NZ, m_slice, n_out), jnp.bfloat16),
            pltpu.SemaphoreType.DMA((NZ - 1,)),
            pltpu.SemaphoreType.DMA((NZ - 1,)),
            pltpu.SemaphoreType.DMA,
            pltpu.SemaphoreType.DMA,
            pltpu.SemaphoreType.DMA((NY - 1,)),
            pltpu.SemaphoreType.DMA((NY - 1,)),
        ],
        compiler_params=pltpu.CompilerParams(collective_id=0),
    )(x)


# device time: 27172 ns/iter; 1.7468x vs baseline; 1.1275x over previous
import jax
import jax.numpy as jnp
from jax import lax
from jax.experimental import pallas as pl
from jax.experimental.pallas import tpu as pltpu

NZ = 4
NY = 4


def kernel(x):
    _, m, n_full = x.shape
    n_out = n_full // NZ
    m_slice = m // 8
    m_block = m // NY

    def body(x_ref, out_ref, zsend_ref, zrecv_ref, zsend_sems, zrecv_sems,
             xsend_sems, xrecv_sems, ysend_sems, yrecv_sems):
        my_x = lax.axis_index("x")
        my_y = lax.axis_index("y")
        my_z = lax.axis_index("z")
        s = my_y * 2 + my_x
        row0 = s * m_slice

        barrier_sem = pltpu.get_barrier_semaphore()
        for j in range(1, NZ):
            pl.semaphore_signal(
                barrier_sem, inc=1,
                device_id=(my_x, my_y, (my_z + j) % NZ),
                device_id_type=pl.DeviceIdType.MESH,
            )
        for j in range(1, NY):
            pl.semaphore_signal(
                barrier_sem, inc=1,
                device_id=(my_x, (my_y + j) % NY, my_z),
                device_id_type=pl.DeviceIdType.MESH,
            )
        pl.semaphore_signal(
            barrier_sem, inc=1,
            device_id=(1 - my_x, my_y, my_z),
            device_id_type=pl.DeviceIdType.MESH,
        )
        pl.semaphore_wait(barrier_sem, NZ - 1 + NY - 1 + 1)

        z_rdmas = []
        for j in range(1, NZ):
            c = (my_z + j) % NZ
            zsend_ref[j - 1, :, :] = x_ref[
                0, pl.ds(row0, m_slice), pl.ds(c * n_out, n_out)
            ].astype(jnp.bfloat16)
            rdma = pltpu.make_async_remote_copy(
                src_ref=zsend_ref.at[j - 1],
                dst_ref=zrecv_ref.at[my_z],
                send_sem=zsend_sems.at[j - 1],
                recv_sem=zrecv_sems.at[j - 1],
                device_id=(my_x, my_y, c),
                device_id_type=pl.DeviceIdType.MESH,
            )
            rdma.start()
            z_rdmas.append(rdma)

        zrecv_ref[my_z, :, :] = x_ref[
            0, pl.ds(row0, m_slice), pl.ds(my_z * n_out, n_out)
        ].astype(jnp.bfloat16)

        for j in range(1, NZ):
            recv = pltpu.make_async_remote_copy(
                src_ref=zsend_ref.at[j - 1],
                dst_ref=zrecv_ref.at[(my_z + NZ - j) % NZ],
                send_sem=zsend_sems.at[j - 1],
                recv_sem=zrecv_sems.at[j - 1],
                device_id=(my_x, my_y, my_z),
                device_id_type=pl.DeviceIdType.MESH,
            )
            recv.wait_recv()

        out_ref[pl.ds(row0, m_slice), :] = (
            zrecv_ref[0] + zrecv_ref[1] + zrecv_ref[2] + zrecv_ref[3]
        )

        y_rdmas = []
        for j in range(1, NY):
            ty = (my_y + j) % NY
            rdma = pltpu.make_async_remote_copy(
                src_ref=out_ref.at[pl.ds(row0, m_slice), :],
                dst_ref=out_ref.at[pl.ds(row0, m_slice), :],
                send_sem=ysend_sems.at[j - 1],
                recv_sem=yrecv_sems.at[j - 1],
                device_id=(my_x, ty, my_z),
                device_id_type=pl.DeviceIdType.MESH,
            )
            rdma.start()
            y_rdmas.append(rdma)

        x_rdmas = []
        x0 = pltpu.make_async_remote_copy(
            src_ref=out_ref.at[pl.ds(row0, m_slice), :],
            dst_ref=out_ref.at[pl.ds(row0, m_slice), :],
            send_sem=xsend_sems.at[0],
            recv_sem=xrecv_sems.at[0],
            device_id=(1 - my_x, my_y, my_z),
            device_id_type=pl.DeviceIdType.MESH,
        )
        x0.start()
        x_rdmas.append(x0)

        for j in range(1, NY):
            oy = (my_y + NY - j) % NY
            orow = (oy * 2 + my_x) * m_slice
            recv = pltpu.make_async_remote_copy(
                src_ref=out_ref.at[pl.ds(row0, m_slice), :],
                dst_ref=out_ref.at[pl.ds(orow, m_slice), :],
                send_sem=ysend_sems.at[j - 1],
                recv_sem=yrecv_sems.at[j - 1],
                device_id=(my_x, my_y, my_z),
                device_id_type=pl.DeviceIdType.MESH,
            )
            recv.wait_recv()
            fwd = pltpu.make_async_remote_copy(
                src_ref=out_ref.at[pl.ds(orow, m_slice), :],
                dst_ref=out_ref.at[pl.ds(orow, m_slice), :],
                send_sem=xsend_sems.at[j],
                recv_sem=xrecv_sems.at[j],
                device_id=(1 - my_x, my_y, my_z),
                device_id_type=pl.DeviceIdType.MESH,
            )
            fwd.start()
            x_rdmas.append(fwd)

        for k in range(NY):
            oy = (my_y + NY - k) % NY
            prow = (oy * 2 + (1 - my_x)) * m_slice
            recv = pltpu.make_async_remote_copy(
                src_ref=out_ref.at[pl.ds(row0, m_slice), :],
                dst_ref=out_ref.at[pl.ds(prow, m_slice), :],
                send_sem=xsend_sems.at[k],
                recv_sem=xrecv_sems.at[k],
                device_id=(my_x, my_y, my_z),
                device_id_type=pl.DeviceIdType.MESH,
            )
            recv.wait_recv()

        for rdma in z_rdmas:
            rdma.wait_send()
        for rdma in y_rdmas:
            rdma.wait_send()
        for rdma in x_rdmas:
            rdma.wait_send()

    return pl.pallas_call(
        body,
        out_shape=jax.ShapeDtypeStruct((m, n_out), jnp.bfloat16),
        in_specs=[pl.BlockSpec(memory_space=pltpu.VMEM)],
        out_specs=pl.BlockSpec(memory_space=pltpu.VMEM),
        scratch_shapes=[
            pltpu.VMEM((NZ - 1, m_slice, n_out), jnp.bfloat16),
            pltpu.VMEM((NZ, m_slice, n_out), jnp.bfloat16),
            pltpu.SemaphoreType.DMA((NZ - 1,)),
            pltpu.SemaphoreType.DMA((NZ - 1,)),
            pltpu.SemaphoreType.DMA((NY,)),
            pltpu.SemaphoreType.DMA((NY,)),
            pltpu.SemaphoreType.DMA((NY - 1,)),
            pltpu.SemaphoreType.DMA((NY - 1,)),
        ],
        compiler_params=pltpu.CompilerParams(collective_id=0),
    )(x)
